# baseline (speedup 1.0000x reference)
import jax
import jax.numpy as jnp
from jax.experimental import pallas as pl
from jax.experimental.pallas import tpu as pltpu


def _body(hs_ref, tv_ref, idx_ref, wd_ref, wu_ref, out_ref, dw_ref):
    bN = hs_ref.shape[0]
    rank = wd_ref.shape[0]
    j = pl.program_id(1)

    @pl.when(j == 0)
    def _compute_down():
        down = jax.lax.dot_general(
            hs_ref[...], wd_ref[...], (((1,), (1,)), ((), ())),
            preferred_element_type=jnp.float32)
        iota = jax.lax.broadcasted_iota(jnp.int32, (bN, rank), 1)
        idx = idx_ref[...]
        tv = tv_ref[...]
        w = jnp.zeros((bN, rank), jnp.float32)
        for k in range(idx.shape[1]):
            w = jnp.where(iota == idx[:, k:k + 1], tv[:, k:k + 1], w)
        dw_ref[...] = down * w

    out_ref[...] = jax.lax.dot_general(
        dw_ref[...], wu_ref[...], (((1,), (1,)), ((), ())),
        preferred_element_type=jnp.float32)


def kernel(hidden_states, top_k_values, top_k_indices, W_down, W_up):
    N, d_in = hidden_states.shape
    rank, _ = W_down.shape
    d_out, _ = W_up.shape
    top_k = top_k_values.shape[1]
    bN = 2048
    bD = 512
    grid = (N // bN, d_out // bD)
    return pl.pallas_call(
        _body,
        grid=grid,
        in_specs=[
            pl.BlockSpec((bN, d_in), lambda i, j: (i, 0)),
            pl.BlockSpec((bN, top_k), lambda i, j: (i, 0)),
            pl.BlockSpec((bN, top_k), lambda i, j: (i, 0)),
            pl.BlockSpec((rank, d_in), lambda i, j: (0, 0)),
            pl.BlockSpec((bD, rank), lambda i, j: (j, 0)),
        ],
        out_specs=pl.BlockSpec((bN, bD), lambda i, j: (i, j)),
        out_shape=jax.ShapeDtypeStruct((N, d_out), jnp.float32),
        scratch_shapes=[pltpu.VMEM((bN, rank), jnp.float32)],
        compiler_params=pltpu.CompilerParams(
            dimension_semantics=("arbitrary", "arbitrary"),
        ),
    )(hidden_states, top_k_values, top_k_indices.astype(jnp.int32),
      W_down, W_up)


# speedup vs baseline: 1.3706x; 1.3706x over previous
import jax
import jax.numpy as jnp
from jax.experimental import pallas as pl
from jax.experimental.pallas import tpu as pltpu

_RC = 1024


def _body(hs_hbm, tv_hbm, idx_hbm, wd_hbm, wu_hbm, out_hbm,
          wd_v, wu_v, hs_v, tv_v, idx_v, out_v, in_sems, w_sem, out_sems):
    g = pl.program_id(0)
    S = pl.num_programs(0)
    s = g % 2

    def start_in(chunk, slot):
        pltpu.make_async_copy(
            hs_hbm.at[pl.ds(chunk * _RC, _RC)], hs_v.at[slot],
            in_sems.at[slot]).start()
        pltpu.make_async_copy(
            tv_hbm.at[pl.ds(chunk * _RC, _RC)], tv_v.at[slot],
            in_sems.at[slot]).start()
        pltpu.make_async_copy(
            idx_hbm.at[pl.ds(chunk * _RC, _RC)], idx_v.at[slot],
            in_sems.at[slot]).start()

    def wait_in(slot):
        pltpu.make_async_copy(
            hs_hbm.at[pl.ds(0, _RC)], hs_v.at[slot], in_sems.at[slot]).wait()
        pltpu.make_async_copy(
            tv_hbm.at[pl.ds(0, _RC)], tv_v.at[slot], in_sems.at[slot]).wait()
        pltpu.make_async_copy(
            idx_hbm.at[pl.ds(0, _RC)], idx_v.at[slot], in_sems.at[slot]).wait()

    def wait_out(chunk, slot):
        pltpu.make_async_copy(
            out_v.at[slot], out_hbm.at[pl.ds(chunk * _RC, _RC)],
            out_sems.at[slot]).wait()

    @pl.when(g == 0)
    def _prologue():
        pltpu.make_async_copy(wd_hbm, wd_v, w_sem).start()
        pltpu.make_async_copy(wu_hbm, wu_v, w_sem).start()
        start_in(0, 0)
        start_in(1, 1)
        pltpu.make_async_copy(wd_hbm, wd_v, w_sem).wait()
        pltpu.make_async_copy(wu_hbm, wu_v, w_sem).wait()

    wait_in(s)

    @pl.when(g >= 2)
    def _recycle_out():
        wait_out(g - 2, s)

    hs = hs_v[s]
    rank = wd_v.shape[0]
    down = jax.lax.dot_general(
        hs, wd_v[...], (((1,), (1,)), ((), ())),
        preferred_element_type=jnp.float32)
    iota = jax.lax.broadcasted_iota(jnp.int32, (_RC, rank), 1)
    idx = idx_v[s]
    tv = tv_v[s]
    w = jnp.zeros((_RC, rank), jnp.float32)
    for k in range(idx.shape[1]):
        w = jnp.where(iota == idx[:, k:k + 1], tv[:, k:k + 1], w)
    out_v[s] = jax.lax.dot_general(
        down * w, wu_v[...], (((1,), (1,)), ((), ())),
        preferred_element_type=jnp.float32)

    pltpu.make_async_copy(
        out_v.at[s], out_hbm.at[pl.ds(g * _RC, _RC)], out_sems.at[s]).start()

    @pl.when(g + 2 < S)
    def _next_in():
        start_in(g + 2, s)

    @pl.when(g == S - 1)
    def _epilogue():
        wait_out(g - 1, (g - 1) % 2)
        wait_out(g, s)


def kernel(hidden_states, top_k_values, top_k_indices, W_down, W_up):
    N, d_in = hidden_states.shape
    rank, _ = W_down.shape
    d_out, _ = W_up.shape
    top_k = top_k_values.shape[1]
    S = N // _RC
    any_spec = pl.BlockSpec(memory_space=pltpu.MemorySpace.HBM)
    return pl.pallas_call(
        _body,
        grid=(S,),
        in_specs=[any_spec] * 5,
        out_specs=any_spec,
        out_shape=jax.ShapeDtypeStruct((N, d_out), jnp.float32),
        scratch_shapes=[
            pltpu.VMEM((rank, d_in), jnp.float32),
            pltpu.VMEM((d_out, rank), jnp.float32),
            pltpu.VMEM((2, _RC, d_in), jnp.float32),
            pltpu.VMEM((2, _RC, top_k), jnp.float32),
            pltpu.VMEM((2, _RC, top_k), jnp.int32),
            pltpu.VMEM((2, _RC, d_out), jnp.float32),
            pltpu.SemaphoreType.DMA((2,)),
            pltpu.SemaphoreType.DMA,
            pltpu.SemaphoreType.DMA((2,)),
        ],
        compiler_params=pltpu.CompilerParams(
            dimension_semantics=("arbitrary",),
        ),
    )(hidden_states, top_k_values, top_k_indices.astype(jnp.int32),
      W_down, W_up)
